# baseline (speedup 1.0000x reference)
import functools
import math

import jax
import jax.numpy as jnp
from jax import lax
from jax.experimental import pallas as pl
from jax.experimental.pallas import tpu as pltpu
from jax.experimental.pallas import tpu_sc as plsc

_N = 50000
_E = 800000
_CW = 64

_NC = 2
_NS = 16
_HALF = _N // 2
_Z = 64
_RPT = ((_HALF + 1 + _NS * _Z - 1) // (_NS * _Z)) * _Z
_R = _RPT * _NS
_C = 128
_SJ = 16
_E2 = 819200
_EROWS = _E2 // _C
_ERPT = _EROWS // _NS
_NSUP = _ERPT // _SJ


def _edge_pass_body(src_hbm, dst_hbm, table_hbm, out_hbm,
                    spbuf, csrc, cdst, ldst, rows0, rows1, zbuf, sem,
                    *, width):
    c = lax.axis_index("c")
    s = lax.axis_index("s")
    half = _HALF

    for r in range(_Z):
        for w in range(width // 16):
            zbuf[r, pl.ds(w * 16, 16)] = jnp.zeros((16,), jnp.float32)
    for z in range(_RPT // _Z):
        pltpu.sync_copy(zbuf, spbuf.at[pl.ds(s * _RPT + z * _Z, _Z)])
    plsc.subcore_barrier()

    rbase = s * _ERPT

    def sup_body(i, carry):
        row = rbase + i * _SJ
        pltpu.sync_copy(src_hbm.at[pl.ds(row, _SJ), :], csrc)
        pltpu.sync_copy(dst_hbm.at[pl.ds(row, _SJ), :], cdst)
        for j in range(_SJ):
            for g in range(8):
                d16 = cdst[j, pl.ds(g * 16, 16)]
                ld = d16 - c * half
                inr = (ld >= 0) & (ld < half)
                ldst[j, pl.ds(g * 16, 16)] = jnp.where(
                    inr, ld, jnp.full((16,), half, jnp.int32))
        bufs = (rows0, rows1)
        pend = pltpu.async_copy(table_hbm.at[csrc.at[0]], bufs[0], sem)
        for j in range(_SJ):
            pend.wait()
            if j + 1 < _SJ:
                pend = pltpu.async_copy(table_hbm.at[csrc.at[j + 1]],
                                        bufs[(j + 1) % 2], sem)
            pltpu.sync_copy(bufs[j % 2], spbuf.at[ldst.at[j]], add=True)
        return carry

    lax.fori_loop(0, _NSUP, sup_body, 0)

    plsc.subcore_barrier()
    for z in range(_RPT // _Z):
        r0 = s * _RPT + z * _Z
        pltpu.sync_copy(spbuf.at[pl.ds(r0, _Z)],
                        out_hbm.at[pl.ds(c * _R + r0, _Z)])


@functools.partial(jax.jit, static_argnames=("width",))
def _edge_pass(src2d, dst2d, table, width):
    mesh = plsc.VectorSubcoreMesh(core_axis_name="c", subcore_axis_name="s")
    body = functools.partial(_edge_pass_body, width=width)
    out = pl.kernel(
        body,
        out_type=jax.ShapeDtypeStruct((_NC * _R, width), jnp.float32),
        mesh=mesh,
        scratch_types=[
            pltpu.VMEM_SHARED((_R, width), jnp.float32),
            pltpu.VMEM((_SJ, _C), jnp.int32),
            pltpu.VMEM((_SJ, _C), jnp.int32),
            pltpu.VMEM((_SJ, _C), jnp.int32),
            pltpu.VMEM((_C, width), jnp.float32),
            pltpu.VMEM((_C, width), jnp.float32),
            pltpu.VMEM((_Z, width), jnp.float32),
            pltpu.SemaphoreType.DMA,
        ],
        compiler_params=pltpu.CompilerParams(use_tc_tiling_on_sc=False),
    )(src2d, dst2d, table)
    return jnp.concatenate([out[:_HALF], out[_R:_R + _HALF]], axis=0)


def _select(h, p, alive, k):
    score = jnp.tanh((h @ p) / jnp.linalg.norm(p))
    key = jnp.where(alive > 0, score, -jnp.inf)
    _, perm = jax.lax.top_k(key, k)
    new_alive = jnp.zeros((_N,), jnp.float32).at[perm].set(1.0)
    return h * score[:, None] * new_alive[:, None], new_alive


def _readout(h, alive, k):
    mx = jnp.max(jnp.where(alive[:, None] > 0, h, -jnp.inf), axis=0,
                 keepdims=True)
    mn = jnp.sum(h, axis=0, keepdims=True) / k
    return jnp.concatenate([mx, mn], axis=1)


def _head_kernel(z, w1, b1, w2, b2, w3, b3, o_ref):
    v = z[...]
    v = jax.nn.relu(v @ w1[...].T + b1[...][None, :])
    v = jax.nn.relu(v @ w2[...].T + b2[...][None, :])
    v = v @ w3[...].T + b3[...][None, :]
    v = v - jnp.max(v, axis=1, keepdims=True)
    e = jnp.exp(v)
    o_ref[...] = e / jnp.sum(e, axis=1, keepdims=True)


def _head(z, w1, b1, w2, b2, w3, b3):
    return pl.pallas_call(
        _head_kernel,
        out_shape=jax.ShapeDtypeStruct((1, 2), jnp.float32),
    )(z, w1, b1, w2, b2, w3, b3)


def kernel(x, edge_index, batch, conv1_Wl, conv1_bl, conv1_Wr, conv2_Wl,
           conv2_bl, conv2_Wr, conv4_W, conv4_b, conv5_W, conv5_b, p1, p2,
           p4, p5, lin1_W, lin1_b, lin2_W, lin2_b, lin3_W, lin3_b):
    src = jnp.pad(edge_index[0], (0, _E2 - _E)).reshape(_EROWS, _C)
    dst = jnp.pad(edge_index[1], (0, _E2 - _E),
                  constant_values=_N).reshape(_EROWS, _C)

    t1 = jnp.concatenate(
        [x, jnp.ones((_N, 1), jnp.float32), jnp.zeros((_N, 13), jnp.float32)],
        axis=1)
    o1 = _edge_pass(src, dst, t1, 16)
    agg = o1[:, :2]
    deg = o1[:, 2]
    mean = agg / jnp.maximum(deg, 1.0)[:, None]
    h = jax.nn.relu(mean @ conv1_Wl.T + conv1_bl + x @ conv1_Wr.T)
    h, alive = _select(h, p1, jnp.ones((_N,), jnp.float32), 25000)
    z = _readout(h, alive, 25000)

    def deg_of(alive_now):
        td = jnp.concatenate(
            [alive_now[:, None], jnp.zeros((_N, 15), jnp.float32)], axis=1)
        return _edge_pass(src, dst, td, 16)[:, 0]

    deg = deg_of(alive)
    agg = _edge_pass(src, dst, h, 64)
    mean = agg / jnp.maximum(deg, 1.0)[:, None]
    h = jax.nn.relu(mean @ conv2_Wl.T + conv2_bl + h @ conv2_Wr.T)
    h, alive = _select(h, p2, alive, 12500)
    z = z + _readout(h, alive, 12500)

    def gcn(h_in, alive_now, Wc, b):
        deg_n = deg_of(alive_now) + 1.0
        dinv = lax.rsqrt(deg_n)
        xw = h_in @ Wc.T
        agg_n = _edge_pass(src, dst, xw * dinv[:, None] * alive_now[:, None],
                           64)
        return agg_n * dinv[:, None] + xw * (dinv * dinv)[:, None] + b

    h = jax.nn.relu(gcn(h, alive, conv4_W, conv4_b))
    h, alive = _select(h, p4, alive, 6250)
    z = z + _readout(h, alive, 6250)

    h = jax.nn.relu(gcn(h, alive, conv5_W, conv5_b))
    h, alive = _select(h, p5, alive, 3125)
    z = z + _readout(h, alive, 3125)

    return _head(z, lin1_W, lin1_b, lin2_W, lin2_b, lin3_W, lin3_b)


# speedup vs baseline: 1.4707x; 1.4707x over previous
import functools
import math

import jax
import jax.numpy as jnp
from jax import lax
from jax.experimental import pallas as pl
from jax.experimental.pallas import tpu as pltpu
from jax.experimental.pallas import tpu_sc as plsc

_N = 50000
_E = 800000
_CW = 64

_NC = 2
_NS = 16
_HALF = _N // 2
_Z = 64
_RPT = ((_HALF + 1 + _NS * _Z - 1) // (_NS * _Z)) * _Z
_R = _RPT * _NS
_C = 128
_SJ = 16
_E2 = 819200
_EROWS = _E2 // _C
_ERPT = _EROWS // _NS
_NSUP = _ERPT // _SJ
_ZC = 16
_RPTC = ((_HALF + 1 + _NS * _ZC - 1) // (_NS * _ZC)) * _ZC
_RC = _RPTC * _NS
_SJC = 8
_NSUPC = _ERPT // _SJC
_ABW = 1568


def _edge_pass_body(src_hbm, dst_hbm, table_hbm, out_hbm,
                    spbuf, csrc, cdst, ldst, rows0, rows1, zbuf, sem,
                    *, width):
    c = lax.axis_index("c")
    s = lax.axis_index("s")
    half = _HALF

    for r in range(_Z):
        for w in range(width // 16):
            zbuf[r, pl.ds(w * 16, 16)] = jnp.zeros((16,), jnp.float32)
    for z in range(_RPT // _Z):
        pltpu.sync_copy(zbuf, spbuf.at[pl.ds(s * _RPT + z * _Z, _Z)])
    plsc.subcore_barrier()

    rbase = s * _ERPT

    def sup_body(i, carry):
        row = rbase + i * _SJ
        pltpu.sync_copy(src_hbm.at[pl.ds(row, _SJ), :], csrc)
        pltpu.sync_copy(dst_hbm.at[pl.ds(row, _SJ), :], cdst)
        for j in range(_SJ):
            for g in range(8):
                d16 = cdst[j, pl.ds(g * 16, 16)]
                ld = d16 - c * half
                inr = (ld >= 0) & (ld < half)
                ldst[j, pl.ds(g * 16, 16)] = jnp.where(
                    inr, ld, jnp.full((16,), half, jnp.int32))
        bufs = (rows0, rows1)
        pend = pltpu.async_copy(table_hbm.at[csrc.at[0]], bufs[0], sem)
        for j in range(_SJ):
            pend.wait()
            if j + 1 < _SJ:
                pend = pltpu.async_copy(table_hbm.at[csrc.at[j + 1]],
                                        bufs[(j + 1) % 2], sem)
            pltpu.sync_copy(bufs[j % 2], spbuf.at[ldst.at[j]], add=True)
        return carry

    lax.fori_loop(0, _NSUP, sup_body, 0)

    plsc.subcore_barrier()
    for z in range(_RPT // _Z):
        r0 = s * _RPT + z * _Z
        pltpu.sync_copy(spbuf.at[pl.ds(r0, _Z)],
                        out_hbm.at[pl.ds(c * _R + r0, _Z)])


def _edge_pass_c_body(src_hbm, dst_hbm, alive_hbm, table_hbm, out_hbm,
                      spbuf, avmem, csrc, cdst, psrc, pdst, psrcF, pdstF,
                      rows0, zbuf, sem, *, width, gather_mode):
    c = lax.axis_index("c")
    s = lax.axis_index("s")
    half = _HALF

    for r in range(_ZC):
        for w in range(width // 16):
            zbuf[r, pl.ds(w * 16, 16)] = jnp.zeros((16,), jnp.float32)
    for z in range(_RPTC // _ZC):
        pltpu.sync_copy(zbuf, spbuf.at[pl.ds(s * _RPTC + z * _ZC, _ZC)])
    pltpu.sync_copy(alive_hbm, avmem)
    if not gather_mode:
        one0 = jnp.where(lax.iota(jnp.int32, 16) == 0, 1.0, 0.0)
        for r in range(_C):
            rows0[r, pl.ds(0, 16)] = one0
    plsc.subcore_barrier()

    def fire():
        for g in range(8):
            psrcF[pl.ds(g * 16, 16)] = psrc[pl.ds(g * 16, 16)]
            pdstF[pl.ds(g * 16, 16)] = pdst[pl.ds(g * 16, 16)]
        if gather_mode:
            pltpu.async_copy(table_hbm.at[psrcF], rows0, sem).wait()
        pltpu.sync_copy(rows0, spbuf.at[pdstF], add=True)
        psrc[pl.ds(0, 16)] = psrc[pl.ds(128, 16)]
        pdst[pl.ds(0, 16)] = pdst[pl.ds(128, 16)]

    rbase = s * _ERPT

    def sup_body(i, P):
        row = rbase + i * _SJC
        pltpu.sync_copy(src_hbm.at[pl.ds(row, _SJC), :], csrc)
        pltpu.sync_copy(dst_hbm.at[pl.ds(row, _SJC), :], cdst)
        for j in range(_SJC):
            for g in range(8):
                s16 = csrc[j, pl.ds(g * 16, 16)]
                d16 = cdst[j, pl.ds(g * 16, 16)]
                ws = plsc.load_gather(avmem, [lax.shift_right_logical(s16, 5)])
                wd = plsc.load_gather(avmem, [lax.shift_right_logical(d16, 5)])
                asrc = lax.shift_right_logical(ws, s16 & 31) & 1
                adst = lax.shift_right_logical(wd, d16 & 31) & 1
                ld = d16 - c * half
                keep = ((asrc > 0) & (adst > 0) & (ld >= 0) & (ld < half))
                ki = jnp.where(keep, 1, 0)
                pos = P + plsc.cumsum(ki) - 1
                plsc.store_scatter(psrc, [pos], s16, mask=keep)
                plsc.store_scatter(pdst, [pos], ld, mask=keep)
                P = P + jnp.sum(ki)
                fire_pred = P >= _C
                pl.when(fire_pred)(fire)
                P = jnp.where(fire_pred, P - _C, P)
        return P

    P = lax.fori_loop(0, _NSUPC, sup_body, jnp.int32(0))

    lane = lax.iota(jnp.int32, 16)
    for g in range(8):
        tr = (lane + g * 16) >= P
        vs = psrc[pl.ds(g * 16, 16)]
        vd = pdst[pl.ds(g * 16, 16)]
        psrc[pl.ds(g * 16, 16)] = jnp.where(tr, 0, vs)
        pdst[pl.ds(g * 16, 16)] = jnp.where(
            tr, jnp.full((16,), half, jnp.int32), vd)
    fire()

    plsc.subcore_barrier()
    for z in range(_RPTC // _ZC):
        r0 = s * _RPTC + z * _ZC
        pltpu.sync_copy(spbuf.at[pl.ds(r0, _ZC)],
                        out_hbm.at[pl.ds(c * _RC + r0, _ZC)])


@functools.partial(jax.jit, static_argnames=("width", "gather_mode"))
def _edge_pass_c(src2d, dst2d, alive_i, table, width, gather_mode):
    mesh = plsc.VectorSubcoreMesh(core_axis_name="c", subcore_axis_name="s")
    body = functools.partial(_edge_pass_c_body, width=width,
                             gather_mode=gather_mode)
    out = pl.kernel(
        body,
        out_type=jax.ShapeDtypeStruct((_NC * _RC, width), jnp.float32),
        mesh=mesh,
        scratch_types=[
            pltpu.VMEM_SHARED((_RC, width), jnp.float32),
            pltpu.VMEM((_ABW,), jnp.int32),
            pltpu.VMEM((_SJC, _C), jnp.int32),
            pltpu.VMEM((_SJC, _C), jnp.int32),
            pltpu.VMEM((144,), jnp.int32),
            pltpu.VMEM((144,), jnp.int32),
            pltpu.VMEM((_C,), jnp.int32),
            pltpu.VMEM((_C,), jnp.int32),
            pltpu.VMEM((_C, width), jnp.float32),
            pltpu.VMEM((_ZC, width), jnp.float32),
            pltpu.SemaphoreType.DMA,
        ],
        compiler_params=pltpu.CompilerParams(use_tc_tiling_on_sc=False,
                                             needs_layout_passes=False),
    )(src2d, dst2d, alive_i, table)
    return jnp.concatenate([out[:_HALF], out[_RC:_RC + _HALF]], axis=0)


@functools.partial(jax.jit, static_argnames=("width",))
def _edge_pass(src2d, dst2d, table, width):
    mesh = plsc.VectorSubcoreMesh(core_axis_name="c", subcore_axis_name="s")
    body = functools.partial(_edge_pass_body, width=width)
    out = pl.kernel(
        body,
        out_type=jax.ShapeDtypeStruct((_NC * _R, width), jnp.float32),
        mesh=mesh,
        scratch_types=[
            pltpu.VMEM_SHARED((_R, width), jnp.float32),
            pltpu.VMEM((_SJ, _C), jnp.int32),
            pltpu.VMEM((_SJ, _C), jnp.int32),
            pltpu.VMEM((_SJ, _C), jnp.int32),
            pltpu.VMEM((_C, width), jnp.float32),
            pltpu.VMEM((_C, width), jnp.float32),
            pltpu.VMEM((_Z, width), jnp.float32),
            pltpu.SemaphoreType.DMA,
        ],
        compiler_params=pltpu.CompilerParams(use_tc_tiling_on_sc=False),
    )(src2d, dst2d, table)
    return jnp.concatenate([out[:_HALF], out[_R:_R + _HALF]], axis=0)


def _select(h, p, alive, k):
    score = jnp.tanh((h @ p) / jnp.linalg.norm(p))
    key = jnp.where(alive > 0, score, -jnp.inf)
    _, perm = jax.lax.top_k(key, k)
    new_alive = jnp.zeros((_N,), jnp.float32).at[perm].set(1.0)
    return h * score[:, None] * new_alive[:, None], new_alive


def _readout(h, alive, k):
    mx = jnp.max(jnp.where(alive[:, None] > 0, h, -jnp.inf), axis=0,
                 keepdims=True)
    mn = jnp.sum(h, axis=0, keepdims=True) / k
    return jnp.concatenate([mx, mn], axis=1)


def _head_kernel(z, w1, b1, w2, b2, w3, b3, o_ref):
    v = z[...]
    v = jax.nn.relu(v @ w1[...].T + b1[...][None, :])
    v = jax.nn.relu(v @ w2[...].T + b2[...][None, :])
    v = v @ w3[...].T + b3[...][None, :]
    v = v - jnp.max(v, axis=1, keepdims=True)
    e = jnp.exp(v)
    o_ref[...] = e / jnp.sum(e, axis=1, keepdims=True)


def _head(z, w1, b1, w2, b2, w3, b3):
    return pl.pallas_call(
        _head_kernel,
        out_shape=jax.ShapeDtypeStruct((1, 2), jnp.float32),
    )(z, w1, b1, w2, b2, w3, b3)


def kernel(x, edge_index, batch, conv1_Wl, conv1_bl, conv1_Wr, conv2_Wl,
           conv2_bl, conv2_Wr, conv4_W, conv4_b, conv5_W, conv5_b, p1, p2,
           p4, p5, lin1_W, lin1_b, lin2_W, lin2_b, lin3_W, lin3_b):
    src = jnp.pad(edge_index[0], (0, _E2 - _E)).reshape(_EROWS, _C)
    dst = jnp.pad(edge_index[1], (0, _E2 - _E),
                  constant_values=_N).reshape(_EROWS, _C)

    t1 = jnp.concatenate(
        [x, jnp.ones((_N, 1), jnp.float32), jnp.zeros((_N, 13), jnp.float32)],
        axis=1)
    o1 = _edge_pass(src, dst, t1, 16)
    agg = o1[:, :2]
    deg = o1[:, 2]
    mean = agg / jnp.maximum(deg, 1.0)[:, None]
    h = jax.nn.relu(mean @ conv1_Wl.T + conv1_bl + x @ conv1_Wr.T)
    h, alive = _select(h, p1, jnp.ones((_N,), jnp.float32), 25000)
    z = _readout(h, alive, 25000)

    def alive_tab(alive_now):
        av = jnp.pad(alive_now.astype(jnp.uint32), (0, _ABW * 32 - _N))
        av = av.reshape(_ABW, 32)
        sh = jnp.arange(32, dtype=jnp.uint32)
        return lax.bitcast_convert_type(
            jnp.sum(av << sh[None, :], axis=1, dtype=jnp.uint32), jnp.int32)

    dummy16 = jnp.zeros((8, 16), jnp.float32)

    ai = alive_tab(alive)
    deg = _edge_pass_c(src, dst, ai, dummy16, 16, False)[:, 0]
    agg = _edge_pass_c(src, dst, ai, h, 64, True)
    mean = agg / jnp.maximum(deg, 1.0)[:, None]
    h = jax.nn.relu(mean @ conv2_Wl.T + conv2_bl + h @ conv2_Wr.T)
    h, alive = _select(h, p2, alive, 12500)
    z = z + _readout(h, alive, 12500)

    def gcn(h_in, alive_now, Wc, b):
        ai_n = alive_tab(alive_now)
        deg_n = _edge_pass_c(src, dst, ai_n, dummy16, 16, False)[:, 0] + 1.0
        dinv = lax.rsqrt(deg_n)
        xw = h_in @ Wc.T
        agg_n = _edge_pass_c(src, dst, ai_n,
                             xw * dinv[:, None] * alive_now[:, None], 64,
                             True)
        return agg_n * dinv[:, None] + xw * (dinv * dinv)[:, None] + b

    h = jax.nn.relu(gcn(h, alive, conv4_W, conv4_b))
    h, alive = _select(h, p4, alive, 6250)
    z = z + _readout(h, alive, 6250)

    h = jax.nn.relu(gcn(h, alive, conv5_W, conv5_b))
    h, alive = _select(h, p5, alive, 3125)
    z = z + _readout(h, alive, 3125)

    return _head(z, lin1_W, lin1_b, lin2_W, lin2_b, lin3_W, lin3_b)
